# baseline (speedup 1.0000x reference)
import functools

import jax
import jax.numpy as jnp
import numpy as np
from jax import lax
from jax.experimental import pallas as pl
from jax.experimental.pallas import tpu as pltpu
from jax.experimental.pallas import tpu_sc as plsc

NUM_FIELDS = 14
VOCAB = 100000
HIDDEN = 64
NUM_NUMERIC = 16
BATCH = 16384

_NC = 2
_NS = 16
_NW = _NC * _NS
_BPW = BATCH // _NW
_BSZ = 64
_NCHUNK = _BPW // _BSZ
_IPC = _BSZ * NUM_FIELDS
_KSUB = _IPC // 128
_IDX_ROWS_PER_CHUNK = _KSUB

_OFFS_NP = ((np.arange(_IPC) % NUM_FIELDS) * VOCAB).astype(np.int32)


def _gather_body(idx_hbm, offs_hbm, table_hbm, out_hbm,
                 idx_v, offs_v, rows_v, acc_v, sem):
    wid = lax.axis_index("s") * _NC + lax.axis_index("c")
    pltpu.sync_copy(offs_hbm, offs_v)
    for g in range(_NCHUNK):
        off0 = wid * (_NCHUNK * _IPC) + g * _IPC
        pltpu.sync_copy(idx_hbm.at[pl.ds(off0, _IPC)], idx_v)
        for t in range(_IPC // 16):
            sl = pl.ds(t * 16, 16)
            idx_v[sl] = idx_v[sl] + offs_v[sl]
        cps = [
            pltpu.async_copy(table_hbm.at[idx_v.at[pl.ds(k * 128, 128)]],
                             rows_v.at[pl.ds(k * 128, 128)], sem)
            for k in range(_KSUB)
        ]
        for c in cps:
            c.wait()
        def _acc(b, carry):
            for s in range(HIDDEN // 16):
                sl = pl.ds(s * 16, 16)
                v = rows_v[b * NUM_FIELDS, sl]
                for i in range(1, NUM_FIELDS):
                    v = v + rows_v[b * NUM_FIELDS + i, sl]
                acc_v[b, sl] = v
            return carry
        lax.fori_loop(0, _BSZ, _acc, 0)
        pltpu.sync_copy(acc_v, out_hbm.at[pl.ds(wid * _BPW + g * _BSZ, _BSZ)])


_gather_sum = functools.partial(
    pl.kernel,
    out_type=jax.ShapeDtypeStruct((BATCH, HIDDEN), jnp.float32),
    mesh=plsc.VectorSubcoreMesh(core_axis_name="c", subcore_axis_name="s"),
    scratch_types=[
        pltpu.VMEM((_IPC,), jnp.int32),
        pltpu.VMEM((_IPC,), jnp.int32),
        pltpu.VMEM((_IPC, HIDDEN), jnp.float32),
        pltpu.VMEM((_BSZ, HIDDEN), jnp.float32),
        pltpu.SemaphoreType.DMA,
    ],
    compiler_params=pltpu.CompilerParams(use_tc_tiling_on_sc=False),
)(_gather_body)


def _mlp_body(emb_ref, xnum_ref, wnum_ref, bnum_ref, w1_ref, b1_ref,
              w2_ref, b2_ref, out_ref):
    x = emb_ref[...] + jnp.dot(xnum_ref[...], wnum_ref[...],
                               preferred_element_type=jnp.float32)
    x = x + bnum_ref[...]
    h = jnp.maximum(jnp.dot(x, w1_ref[...],
                            preferred_element_type=jnp.float32) + b1_ref[...],
                    0.0)
    out_ref[...] = jnp.dot(h, w2_ref[...],
                           preferred_element_type=jnp.float32) + b2_ref[...]


_MLP_BB = 2048


def _mlp(emb, x_num, W_num, b_num, W1, b1, W2, b2):
    grid = (BATCH // _MLP_BB,)
    full = lambda shape: pl.BlockSpec(shape, lambda i: (0, 0))
    return pl.pallas_call(
        _mlp_body,
        grid=grid,
        in_specs=[
            pl.BlockSpec((_MLP_BB, HIDDEN), lambda i: (i, 0)),
            pl.BlockSpec((_MLP_BB, NUM_NUMERIC), lambda i: (i, 0)),
            full((NUM_NUMERIC, HIDDEN)),
            full((1, HIDDEN)),
            full((HIDDEN, 64)),
            full((1, 64)),
            full((64, 2)),
            full((1, 2)),
        ],
        out_specs=pl.BlockSpec((_MLP_BB, 2), lambda i: (i, 0)),
        out_shape=jax.ShapeDtypeStruct((BATCH, 2), jnp.float32),
    )(emb, x_num, W_num, b_num, W1, b1, W2, b2)


def kernel(x_cat, x_num, emb_tables, W_num, b_num, W1, b1, W2, b2):
    table = emb_tables.reshape(NUM_FIELDS * VOCAB, HIDDEN)
    idx1d = x_cat.astype(jnp.int32).reshape(-1)
    offs = jnp.asarray(_OFFS_NP)
    emb = _gather_sum(idx1d, offs, table)
    return _mlp(emb, x_num, W_num, b_num.reshape(1, -1), W1,
                b1.reshape(1, -1), W2, b2.reshape(1, -1))
